# baseline (speedup 1.0000x reference)
import jax
import jax.numpy as jnp
from jax import lax
from jax.experimental import pallas as pl
from jax.experimental.pallas import tpu as pltpu
from jax.experimental.pallas import tpu_sc as plsc

N = 10000
NPAD = 10240
E = 320000
NC, NS = 2, 16
NW = NC * NS
K = 128
NCH = E // K
CPW = NCH // NW
NEPI = NCH - CPW * NW
NBUF = 13
RPT = NPAD // NS
BLK = 10000


def _sc_agg(D, NBUF):
    mesh = plsc.VectorSubcoreMesh(core_axis_name="c", subcore_axis_name="s")

    def body(y, src2d, dst2d, zrow, out, srcb, dstb, rows, agg,
             si_sem, di_sem, *gsems):
        c = lax.axis_index("c")
        s = lax.axis_index("s")
        wid = s * NC + c
        rbase = s * RPT
        cbase = wid * CPW

        pltpu.async_copy(src2d.at[pl.ds(cbase, CPW)],
                         srcb.at[pl.ds(0, CPW)], si_sem)
        pltpu.async_copy(dst2d.at[pl.ds(cbase, CPW)],
                         dstb.at[pl.ds(0, CPW)], di_sem)

        @pl.when(wid < NEPI)
        def _():
            pltpu.async_copy(src2d.at[pl.ds(NW * CPW + wid, 1)],
                             srcb.at[pl.ds(CPW, 1)], si_sem)
            pltpu.async_copy(dst2d.at[pl.ds(NW * CPW + wid, 1)],
                             dstb.at[pl.ds(CPW, 1)], di_sem)

        pltpu.sync_copy(zrow, rows.at[0])
        for q in range(RPT // K):
            pltpu.sync_copy(rows.at[0], agg.at[pl.ds(rbase + q * K, K)])
        plsc.subcore_barrier()

        pltpu.make_async_copy(src2d.at[pl.ds(cbase, CPW)],
                              srcb.at[pl.ds(0, CPW)], si_sem).wait()
        pltpu.make_async_copy(dst2d.at[pl.ds(cbase, CPW)],
                              dstb.at[pl.ds(0, CPW)], di_sem).wait()

        @pl.when(wid < NEPI)
        def _():
            pltpu.make_async_copy(src2d.at[pl.ds(NW * CPW + wid, 1)],
                                  srcb.at[pl.ds(CPW, 1)], si_sem).wait()
            pltpu.make_async_copy(dst2d.at[pl.ds(NW * CPW + wid, 1)],
                                  dstb.at[pl.ds(CPW, 1)], di_sem).wait()

        def gstart(j, b):
            pltpu.async_copy(y.at[srcb.at[j]], rows.at[b], gsems[b])

        def gwait(j, b):
            pltpu.make_async_copy(y.at[srcb.at[j]], rows.at[b],
                                  gsems[b]).wait()

        for b in range(NBUF):
            gstart(b, b)

        def loop_body(i, carry):
            for b in range(NBUF):
                j = i * NBUF + b
                gwait(j, b)
                pltpu.sync_copy(rows.at[b], agg.at[dstb.at[j]], add=True)

                @pl.when(i < CPW // NBUF - 1)
                def _():
                    gstart(j + NBUF, b)

            return carry

        lax.fori_loop(0, CPW // NBUF, loop_body, 0)

        @pl.when(wid < NEPI)
        def _():
            gstart(CPW, 0)
            gwait(CPW, 0)
            pltpu.sync_copy(rows.at[0], agg.at[dstb.at[CPW]], add=True)

        plsc.subcore_barrier()
        pltpu.sync_copy(agg.at[pl.ds(rbase, RPT)],
                        out.at[c, pl.ds(rbase, RPT)])

    return pl.kernel(
        body,
        out_type=jax.ShapeDtypeStruct((NC, NPAD, D), jnp.float32),
        mesh=mesh,
        scratch_types=[
            pltpu.VMEM((CPW + 1, K), jnp.int32),
            pltpu.VMEM((CPW + 1, K), jnp.int32),
            pltpu.VMEM((NBUF, K, D), jnp.float32),
            pltpu.VMEM_SHARED((NPAD, D), jnp.float32),
        ] + [pltpu.SemaphoreType.DMA] * (2 + NBUF),
        compiler_params=pltpu.CompilerParams(use_tc_tiling_on_sc=False),
    )


def _sc_deg():
    mesh = plsc.VectorSubcoreMesh(core_axis_name="c", subcore_axis_name="s")

    def body(dst2d, ones1, zeros1, out, dstb, onesb, agg, di_sem, ssem):
        c = lax.axis_index("c")
        s = lax.axis_index("s")
        wid = s * NC + c
        rbase = s * RPT
        cbase = wid * CPW

        pltpu.async_copy(dst2d.at[pl.ds(cbase, CPW)],
                         dstb.at[pl.ds(0, CPW)], di_sem)

        @pl.when(wid < NEPI)
        def _():
            pltpu.async_copy(dst2d.at[pl.ds(NW * CPW + wid, 1)],
                             dstb.at[pl.ds(CPW, 1)], di_sem)

        pltpu.sync_copy(ones1, onesb)
        ones_rows = onesb.at[pl.ds(0, K)]

        @pl.when(c == 0)
        def _():
            for q in range(RPT // K):
                pltpu.sync_copy(ones1, agg.at[pl.ds(rbase + q * K, K)])

        @pl.when(c != 0)
        def _():
            for q in range(RPT // K):
                pltpu.sync_copy(zeros1, agg.at[pl.ds(rbase + q * K, K)])

        plsc.subcore_barrier()

        pltpu.make_async_copy(dst2d.at[pl.ds(cbase, CPW)],
                              dstb.at[pl.ds(0, CPW)], di_sem).wait()

        @pl.when(wid < NEPI)
        def _():
            pltpu.make_async_copy(dst2d.at[pl.ds(NW * CPW + wid, 1)],
                                  dstb.at[pl.ds(CPW, 1)], di_sem).wait()

        def loop_body(i, carry):
            for b in range(NBUF):
                pltpu.async_copy(ones_rows, agg.at[dstb.at[i * NBUF + b]],
                                 ssem, add=True)
            for b in range(NBUF):
                pltpu.make_async_copy(ones_rows,
                                      agg.at[dstb.at[i * NBUF + b]],
                                      ssem).wait()
            return carry

        lax.fori_loop(0, CPW // NBUF, loop_body, 0)

        @pl.when(wid < NEPI)
        def _():
            pltpu.sync_copy(ones_rows, agg.at[dstb.at[CPW]], add=True)

        plsc.subcore_barrier()
        pltpu.sync_copy(agg.at[pl.ds(rbase, RPT)],
                        out.at[c, pl.ds(rbase, RPT)])

    return pl.kernel(
        body,
        out_type=jax.ShapeDtypeStruct((NC, NPAD, 1), jnp.float32),
        mesh=mesh,
        scratch_types=[
            pltpu.VMEM((CPW + 1, K), jnp.int32),
            pltpu.VMEM((K, 1), jnp.float32),
            pltpu.VMEM_SHARED((NPAD, 1), jnp.float32),
        ] + [pltpu.SemaphoreType.DMA] * 2,
        compiler_params=pltpu.CompilerParams(use_tc_tiling_on_sc=False),
    )


def _sc_final():
    mesh = plsc.VectorSubcoreMesh(core_axis_name="c", subcore_axis_name="s")
    RPW = NPAD // NW

    def body(p3t, y3t, disf, consts, out, p0b, p1b, y3b, disb, cb, outb, sem):
        c = lax.axis_index("c")
        s = lax.axis_index("s")
        wid = s * NC + c
        r0 = jnp.minimum(wid * RPW, N - RPW)

        def dmas():
            ops = []
            for k in range(16):
                ops.append((p3t.at[0, pl.ds(k * NPAD + r0, RPW)],
                            p0b.at[pl.ds(k * RPW, RPW)]))
                ops.append((p3t.at[1, pl.ds(k * NPAD + r0, RPW)],
                            p1b.at[pl.ds(k * RPW, RPW)]))
                ops.append((y3t.at[pl.ds(k * N + r0, RPW)],
                            y3b.at[pl.ds(k * RPW, RPW)]))
            ops.append((disf.at[pl.ds(r0, RPW)], disb))
            ops.append((consts, cb))
            return ops

        for src_, dst in dmas():
            pltpu.async_copy(src_, dst, sem)
        for src_, dst in dmas():
            pltpu.make_async_copy(src_, dst, sem).wait()

        def group(g, carry):
            o = g * 16
            dis16 = disb[pl.ds(o, 16)]
            acc = cb[pl.ds(512, 16)]
            for k in range(16):
                col = (p0b[pl.ds(k * RPW + o, 16)]
                       + p1b[pl.ds(k * RPW + o, 16)]
                       + y3b[pl.ds(k * RPW + o, 16)])
                h = jnp.maximum(dis16 * col + cb[pl.ds(k * 16, 16)], 0.0)
                acc = acc + h * cb[pl.ds(256 + k * 16, 16)]
            outb[pl.ds(o, 16)] = acc
            return carry

        lax.fori_loop(0, RPW // 16, group, 0)
        pltpu.sync_copy(outb, out.at[pl.ds(r0, RPW)])

    return pl.kernel(
        body,
        out_type=jax.ShapeDtypeStruct((NPAD,), jnp.float32),
        mesh=mesh,
        scratch_types=[
            pltpu.VMEM((16 * RPW,), jnp.float32),
            pltpu.VMEM((16 * RPW,), jnp.float32),
            pltpu.VMEM((16 * RPW,), jnp.float32),
            pltpu.VMEM((RPW,), jnp.float32),
            pltpu.VMEM((528,), jnp.float32),
            pltpu.VMEM((RPW,), jnp.float32),
            pltpu.SemaphoreType.DMA,
        ],
        compiler_params=pltpu.CompilerParams(use_tc_tiling_on_sc=False),
    )


_deg = _sc_deg()
_agg64 = _sc_agg(64, 6)
_agg32 = _sc_agg(32, 13)
_agg16 = _sc_agg(16, 13)
_final = _sc_final()


def _tc_prep(degp, x, W1):
    def body(degp_ref, x_ref, w_ref, y_ref, dis_ref):
        deg = degp_ref[0] + degp_ref[1]
        dis = lax.rsqrt(deg)
        dis_ref[...] = dis
        y_ref[...] = dis * jnp.dot(x_ref[...], w_ref[...],
                                   preferred_element_type=jnp.float32)

    return pl.pallas_call(
        body,
        grid=(N // BLK,),
        in_specs=[
            pl.BlockSpec((NC, BLK, 1), lambda i: (0, i, 0)),
            pl.BlockSpec((BLK, 128), lambda i: (i, 0)),
            pl.BlockSpec((128, 64), lambda i: (0, 0)),
        ],
        out_specs=[
            pl.BlockSpec((BLK, 64), lambda i: (i, 0)),
            pl.BlockSpec((BLK, 1), lambda i: (i, 0)),
        ],
        out_shape=[
            jax.ShapeDtypeStruct((N, 64), jnp.float32),
            jax.ShapeDtypeStruct((N, 1), jnp.float32),
        ],
    )(degp, x, W1)


def _tc_mid(p, y, dis, b, W, Din, Dout):
    def body(p_ref, y_ref, dis_ref, b_ref, w_ref, o_ref):
        h = jnp.maximum(
            dis_ref[...] * (p_ref[0] + p_ref[1] + y_ref[...]) + b_ref[...],
            0.0)
        o_ref[...] = dis_ref[...] * jnp.dot(h, w_ref[...],
                                            preferred_element_type=jnp.float32)

    return pl.pallas_call(
        body,
        grid=(N // BLK,),
        in_specs=[
            pl.BlockSpec((NC, BLK, Din), lambda i: (0, i, 0)),
            pl.BlockSpec((BLK, Din), lambda i: (i, 0)),
            pl.BlockSpec((BLK, 1), lambda i: (i, 0)),
            pl.BlockSpec((1, Din), lambda i: (0, 0)),
            pl.BlockSpec((Din, Dout), lambda i: (0, 0)),
        ],
        out_specs=pl.BlockSpec((BLK, Dout), lambda i: (i, 0)),
        out_shape=jax.ShapeDtypeStruct((N, Dout), jnp.float32),
    )(p, y, dis, b, W)


def _tc_final(p, y, dis, b3, Wout, bout):
    def body(p_ref, y_ref, dis_ref, b_ref, w_ref, bo_ref, o_ref):
        h = jnp.maximum(
            dis_ref[...] * (p_ref[0] + p_ref[1] + y_ref[...]) + b_ref[...],
            0.0)
        o_ref[...] = jnp.dot(h, w_ref[...],
                             preferred_element_type=jnp.float32) + bo_ref[...]

    return pl.pallas_call(
        body,
        grid=(N // BLK,),
        in_specs=[
            pl.BlockSpec((NC, BLK, 16), lambda i: (0, i, 0)),
            pl.BlockSpec((BLK, 16), lambda i: (i, 0)),
            pl.BlockSpec((BLK, 1), lambda i: (i, 0)),
            pl.BlockSpec((1, 16), lambda i: (0, 0)),
            pl.BlockSpec((16, 1), lambda i: (0, 0)),
            pl.BlockSpec((1, 1), lambda i: (0, 0)),
        ],
        out_specs=pl.BlockSpec((BLK, 1), lambda i: (i, 0)),
        out_shape=jax.ShapeDtypeStruct((N, 1), jnp.float32),
    )(p, y, dis, b3, Wout, bout)


def kernel(x, edge_index, W1, b1, W2, b2, W3, b3, Wout, bout):
    ei = edge_index.astype(jnp.int32)
    src2d = ei[0].reshape(NCH, K)
    dst2d = ei[1].reshape(NCH, K)
    ones1 = jnp.ones((K, 1), jnp.float32)
    zeros1 = jnp.zeros((K, 1), jnp.float32)
    zeros64 = jnp.zeros((K, 64), jnp.float32)

    degp = _deg(dst2d, ones1, zeros1)
    y1, dis = _tc_prep(degp, x, W1)
    p1 = _agg64(y1, src2d, dst2d, zeros64)
    y2 = _tc_mid(p1, y1, dis, b1.reshape(1, -1), W2, 64, 32)
    p2 = _agg32(y2, src2d, dst2d, zeros64[:, :32])
    y3 = _tc_mid(p2, y2, dis, b2.reshape(1, -1), W3, 32, 16)
    p3 = _agg16(y3, src2d, dst2d, zeros64[:, :16])
    consts = jnp.concatenate(
        [jnp.repeat(b3, 16), jnp.repeat(Wout.reshape(-1), 16),
         jnp.broadcast_to(bout, (16,))])
    outp = _final(p3.transpose(0, 2, 1).reshape(NC, 16 * NPAD),
                  y3.T.reshape(-1), dis.reshape(-1), consts)
    return outp[:N].reshape(N, 1)


# speedup vs baseline: 1.0528x; 1.0528x over previous
import jax
import jax.numpy as jnp
from jax import lax
from jax.experimental import pallas as pl
from jax.experimental.pallas import tpu as pltpu
from jax.experimental.pallas import tpu_sc as plsc

N = 10000
NPAD = 10240
E = 320000
NC, NS = 2, 16
NW = NC * NS
K = 128
NCH = E // K
CPW = NCH // NW
NEPI = NCH - CPW * NW
NBUF = 13
RPT = NPAD // NS
BLK = 10000


def _sc_agg(D, NBUF):
    mesh = plsc.VectorSubcoreMesh(core_axis_name="c", subcore_axis_name="s")

    def body(y, ei3, zrow, out, srcb, dstb, rows, agg,
             si_sem, di_sem, *gsems):
        c = lax.axis_index("c")
        s = lax.axis_index("s")
        wid = s * NC + c
        rbase = s * RPT
        cbase = wid * CPW

        pltpu.async_copy(ei3.at[0, pl.ds(cbase, CPW)],
                         srcb.at[pl.ds(0, CPW)], si_sem)
        pltpu.async_copy(ei3.at[1, pl.ds(cbase, CPW)],
                         dstb.at[pl.ds(0, CPW)], di_sem)

        @pl.when(wid < NEPI)
        def _():
            pltpu.async_copy(ei3.at[0, pl.ds(NW * CPW + wid, 1)],
                             srcb.at[pl.ds(CPW, 1)], si_sem)
            pltpu.async_copy(ei3.at[1, pl.ds(NW * CPW + wid, 1)],
                             dstb.at[pl.ds(CPW, 1)], di_sem)

        pltpu.sync_copy(zrow, rows.at[0])
        for q in range(RPT // K):
            pltpu.sync_copy(rows.at[0], agg.at[pl.ds(rbase + q * K, K)])
        plsc.subcore_barrier()

        pltpu.make_async_copy(ei3.at[0, pl.ds(cbase, CPW)],
                              srcb.at[pl.ds(0, CPW)], si_sem).wait()
        pltpu.make_async_copy(ei3.at[1, pl.ds(cbase, CPW)],
                              dstb.at[pl.ds(0, CPW)], di_sem).wait()

        @pl.when(wid < NEPI)
        def _():
            pltpu.make_async_copy(ei3.at[0, pl.ds(NW * CPW + wid, 1)],
                                  srcb.at[pl.ds(CPW, 1)], si_sem).wait()
            pltpu.make_async_copy(ei3.at[1, pl.ds(NW * CPW + wid, 1)],
                                  dstb.at[pl.ds(CPW, 1)], di_sem).wait()

        def gstart(j, b):
            pltpu.async_copy(y.at[srcb.at[j]], rows.at[b], gsems[b])

        def gwait(j, b):
            pltpu.make_async_copy(y.at[srcb.at[j]], rows.at[b],
                                  gsems[b]).wait()

        for b in range(NBUF):
            gstart(b, b)

        def loop_body(i, carry):
            for b in range(NBUF):
                j = i * NBUF + b
                gwait(j, b)
                pltpu.sync_copy(rows.at[b], agg.at[dstb.at[j]], add=True)

                @pl.when(i < CPW // NBUF - 1)
                def _():
                    gstart(j + NBUF, b)

            return carry

        lax.fori_loop(0, CPW // NBUF, loop_body, 0)

        @pl.when(wid < NEPI)
        def _():
            gstart(CPW, 0)
            gwait(CPW, 0)
            pltpu.sync_copy(rows.at[0], agg.at[dstb.at[CPW]], add=True)

        plsc.subcore_barrier()
        pltpu.sync_copy(agg.at[pl.ds(rbase, RPT)],
                        out.at[c, pl.ds(rbase, RPT)])

    return pl.kernel(
        body,
        out_type=jax.ShapeDtypeStruct((NC, NPAD, D), jnp.float32),
        mesh=mesh,
        scratch_types=[
            pltpu.VMEM((CPW + 1, K), jnp.int32),
            pltpu.VMEM((CPW + 1, K), jnp.int32),
            pltpu.VMEM((NBUF, K, D), jnp.float32),
            pltpu.VMEM_SHARED((NPAD, D), jnp.float32),
        ] + [pltpu.SemaphoreType.DMA] * (2 + NBUF),
        compiler_params=pltpu.CompilerParams(use_tc_tiling_on_sc=False),
    )


def _sc_deg():
    mesh = plsc.VectorSubcoreMesh(core_axis_name="c", subcore_axis_name="s")

    def body(ei3, ones1, zeros1, out, dstb, onesb, agg, di_sem, ssem):
        c = lax.axis_index("c")
        s = lax.axis_index("s")
        wid = s * NC + c
        rbase = s * RPT
        cbase = wid * CPW

        pltpu.async_copy(ei3.at[1, pl.ds(cbase, CPW)],
                         dstb.at[pl.ds(0, CPW)], di_sem)

        @pl.when(wid < NEPI)
        def _():
            pltpu.async_copy(ei3.at[1, pl.ds(NW * CPW + wid, 1)],
                             dstb.at[pl.ds(CPW, 1)], di_sem)

        pltpu.sync_copy(ones1, onesb)
        ones_rows = onesb.at[pl.ds(0, K)]

        @pl.when(c == 0)
        def _():
            for q in range(RPT // K):
                pltpu.sync_copy(ones1, agg.at[pl.ds(rbase + q * K, K)])

        @pl.when(c != 0)
        def _():
            for q in range(RPT // K):
                pltpu.sync_copy(zeros1, agg.at[pl.ds(rbase + q * K, K)])

        plsc.subcore_barrier()

        pltpu.make_async_copy(ei3.at[1, pl.ds(cbase, CPW)],
                              dstb.at[pl.ds(0, CPW)], di_sem).wait()

        @pl.when(wid < NEPI)
        def _():
            pltpu.make_async_copy(ei3.at[1, pl.ds(NW * CPW + wid, 1)],
                                  dstb.at[pl.ds(CPW, 1)], di_sem).wait()

        def loop_body(i, carry):
            for b in range(NBUF):
                pltpu.async_copy(ones_rows, agg.at[dstb.at[i * NBUF + b]],
                                 ssem, add=True)
            for b in range(NBUF):
                pltpu.make_async_copy(ones_rows,
                                      agg.at[dstb.at[i * NBUF + b]],
                                      ssem).wait()
            return carry

        lax.fori_loop(0, CPW // NBUF, loop_body, 0)

        @pl.when(wid < NEPI)
        def _():
            pltpu.sync_copy(ones_rows, agg.at[dstb.at[CPW]], add=True)

        plsc.subcore_barrier()
        pltpu.sync_copy(agg.at[pl.ds(rbase, RPT)],
                        out.at[c, pl.ds(rbase, RPT)])

    return pl.kernel(
        body,
        out_type=jax.ShapeDtypeStruct((NC, NPAD, 1), jnp.float32),
        mesh=mesh,
        scratch_types=[
            pltpu.VMEM((CPW + 1, K), jnp.int32),
            pltpu.VMEM((K, 1), jnp.float32),
            pltpu.VMEM_SHARED((NPAD, 1), jnp.float32),
        ] + [pltpu.SemaphoreType.DMA] * 2,
        compiler_params=pltpu.CompilerParams(use_tc_tiling_on_sc=False),
    )


def _sc_final():
    mesh = plsc.VectorSubcoreMesh(core_axis_name="c", subcore_axis_name="s")
    RPW = NPAD // NW

    def body(p3t, y3t, disf, consts, out, p0b, p1b, y3b, disb, cb, outb, sem):
        c = lax.axis_index("c")
        s = lax.axis_index("s")
        wid = s * NC + c
        r0 = jnp.minimum(wid * RPW, N - RPW)

        def dmas():
            ops = []
            for k in range(16):
                ops.append((p3t.at[0, pl.ds(k * NPAD + r0, RPW)],
                            p0b.at[pl.ds(k * RPW, RPW)]))
                ops.append((p3t.at[1, pl.ds(k * NPAD + r0, RPW)],
                            p1b.at[pl.ds(k * RPW, RPW)]))
                ops.append((y3t.at[pl.ds(k * N + r0, RPW)],
                            y3b.at[pl.ds(k * RPW, RPW)]))
            ops.append((disf.at[pl.ds(r0, RPW)], disb))
            ops.append((consts, cb))
            return ops

        for src_, dst in dmas():
            pltpu.async_copy(src_, dst, sem)
        for src_, dst in dmas():
            pltpu.make_async_copy(src_, dst, sem).wait()

        def group(g, carry):
            o = g * 16
            dis16 = disb[pl.ds(o, 16)]
            acc = cb[pl.ds(512, 16)]
            for k in range(16):
                col = (p0b[pl.ds(k * RPW + o, 16)]
                       + p1b[pl.ds(k * RPW + o, 16)]
                       + y3b[pl.ds(k * RPW + o, 16)])
                h = jnp.maximum(dis16 * col + cb[pl.ds(k * 16, 16)], 0.0)
                acc = acc + h * cb[pl.ds(256 + k * 16, 16)]
            outb[pl.ds(o, 16)] = acc
            return carry

        lax.fori_loop(0, RPW // 16, group, 0)
        pltpu.sync_copy(outb, out.at[pl.ds(r0, RPW)])

    return pl.kernel(
        body,
        out_type=jax.ShapeDtypeStruct((NPAD,), jnp.float32),
        mesh=mesh,
        scratch_types=[
            pltpu.VMEM((16 * RPW,), jnp.float32),
            pltpu.VMEM((16 * RPW,), jnp.float32),
            pltpu.VMEM((16 * RPW,), jnp.float32),
            pltpu.VMEM((RPW,), jnp.float32),
            pltpu.VMEM((528,), jnp.float32),
            pltpu.VMEM((RPW,), jnp.float32),
            pltpu.SemaphoreType.DMA,
        ],
        compiler_params=pltpu.CompilerParams(use_tc_tiling_on_sc=False),
    )


_deg = _sc_deg()
_agg64 = _sc_agg(64, 6)
_agg32 = _sc_agg(32, 13)
_agg16 = _sc_agg(16, 13)
_final = _sc_final()


def _tc_prep(degp, x, W1):
    def body(degp_ref, x_ref, w_ref, y_ref, dis_ref):
        deg = degp_ref[0] + degp_ref[1]
        dis = lax.rsqrt(deg)
        dis_ref[...] = dis
        y_ref[...] = dis * jnp.dot(x_ref[...], w_ref[...],
                                   preferred_element_type=jnp.float32)

    return pl.pallas_call(
        body,
        grid=(N // BLK,),
        in_specs=[
            pl.BlockSpec((NC, BLK, 1), lambda i: (0, i, 0)),
            pl.BlockSpec((BLK, 128), lambda i: (i, 0)),
            pl.BlockSpec((128, 64), lambda i: (0, 0)),
        ],
        out_specs=[
            pl.BlockSpec((BLK, 64), lambda i: (i, 0)),
            pl.BlockSpec((BLK, 1), lambda i: (i, 0)),
        ],
        out_shape=[
            jax.ShapeDtypeStruct((N, 64), jnp.float32),
            jax.ShapeDtypeStruct((N, 1), jnp.float32),
        ],
    )(degp, x, W1)


def _tc_mid(p, y, dis, b, W, Din, Dout):
    def body(p_ref, y_ref, dis_ref, b_ref, w_ref, o_ref):
        h = jnp.maximum(
            dis_ref[...] * (p_ref[0] + p_ref[1] + y_ref[...]) + b_ref[...],
            0.0)
        o_ref[...] = dis_ref[...] * jnp.dot(h, w_ref[...],
                                            preferred_element_type=jnp.float32)

    return pl.pallas_call(
        body,
        grid=(N // BLK,),
        in_specs=[
            pl.BlockSpec((NC, BLK, Din), lambda i: (0, i, 0)),
            pl.BlockSpec((BLK, Din), lambda i: (i, 0)),
            pl.BlockSpec((BLK, 1), lambda i: (i, 0)),
            pl.BlockSpec((1, Din), lambda i: (0, 0)),
            pl.BlockSpec((Din, Dout), lambda i: (0, 0)),
        ],
        out_specs=pl.BlockSpec((BLK, Dout), lambda i: (i, 0)),
        out_shape=jax.ShapeDtypeStruct((N, Dout), jnp.float32),
    )(p, y, dis, b, W)


def _tc_final(p, y, dis, b3, Wout, bout):
    def body(p_ref, y_ref, dis_ref, b_ref, w_ref, bo_ref, o_ref):
        h = jnp.maximum(
            dis_ref[...] * (p_ref[0] + p_ref[1] + y_ref[...]) + b_ref[...],
            0.0)
        o_ref[...] = jnp.dot(h, w_ref[...],
                             preferred_element_type=jnp.float32) + bo_ref[...]

    return pl.pallas_call(
        body,
        grid=(N // BLK,),
        in_specs=[
            pl.BlockSpec((NC, BLK, 16), lambda i: (0, i, 0)),
            pl.BlockSpec((BLK, 16), lambda i: (i, 0)),
            pl.BlockSpec((BLK, 1), lambda i: (i, 0)),
            pl.BlockSpec((1, 16), lambda i: (0, 0)),
            pl.BlockSpec((16, 1), lambda i: (0, 0)),
            pl.BlockSpec((1, 1), lambda i: (0, 0)),
        ],
        out_specs=pl.BlockSpec((BLK, 1), lambda i: (i, 0)),
        out_shape=jax.ShapeDtypeStruct((N, 1), jnp.float32),
    )(p, y, dis, b3, Wout, bout)


def kernel(x, edge_index, W1, b1, W2, b2, W3, b3, Wout, bout):
    ei3 = edge_index.astype(jnp.int32).reshape(2, NCH, K)
    ones1 = jnp.ones((K, 1), jnp.float32)
    zeros1 = jnp.zeros((K, 1), jnp.float32)
    zeros64 = jnp.zeros((K, 64), jnp.float32)

    degp = _deg(ei3, ones1, zeros1)
    y1, dis = _tc_prep(degp, x, W1)
    p1 = _agg64(y1, ei3, zeros64)
    y2 = _tc_mid(p1, y1, dis, b1.reshape(1, -1), W2, 64, 32)
    p2 = _agg32(y2, ei3, zeros64[:, :32])
    y3 = _tc_mid(p2, y2, dis, b2.reshape(1, -1), W3, 32, 16)
    p3 = _agg16(y3, ei3, zeros64[:, :16])
    consts = jnp.concatenate(
        [jnp.repeat(b3, 16), jnp.repeat(Wout.reshape(-1), 16),
         jnp.broadcast_to(bout, (16,))])
    outp = _final(p3.transpose(0, 2, 1).reshape(NC, 16 * NPAD),
                  y3.T.reshape(-1), dis.reshape(-1), consts)
    return outp[:N].reshape(N, 1)
